# baseline (speedup 1.0000x reference)
import jax
import jax.numpy as jnp
from jax.experimental import pallas as pl


def _stable_zero_div_body(x_ref, y_ref, o_ref):
    x = x_ref[...]
    y = y_ref[...]
    nz = y != 0.0
    inv = jnp.where(nz, 1.0 / jnp.where(nz, y, 1.0), 0.0)
    o_ref[...] = inv * x


def kernel(x, y):
    n = x.shape[0]
    cols = 1024
    rows = n // cols
    block_rows = 512
    x2 = x.reshape(rows, cols)
    y2 = y.reshape(rows, cols)
    out = pl.pallas_call(
        _stable_zero_div_body,
        grid=(rows // block_rows,),
        in_specs=[
            pl.BlockSpec((block_rows, cols), lambda i: (i, 0)),
            pl.BlockSpec((block_rows, cols), lambda i: (i, 0)),
        ],
        out_specs=pl.BlockSpec((block_rows, cols), lambda i: (i, 0)),
        out_shape=jax.ShapeDtypeStruct((rows, cols), jnp.float32),
    )(x2, y2)
    return out.reshape(n)


# speedup vs baseline: 4.1024x; 4.1024x over previous
import jax
import jax.numpy as jnp
from jax.experimental import pallas as pl


def _stable_zero_div_body(x_ref, y_ref, o_ref):
    x = x_ref[...]
    y = y_ref[...]
    nz = y != 0.0
    inv = jnp.where(nz, 1.0 / jnp.where(nz, y, 1.0), 0.0)
    o_ref[...] = inv * x


def kernel(x, y):
    n = x.shape[0]
    block = 524288
    out = pl.pallas_call(
        _stable_zero_div_body,
        grid=(n // block,),
        in_specs=[
            pl.BlockSpec((block,), lambda i: (i,)),
            pl.BlockSpec((block,), lambda i: (i,)),
        ],
        out_specs=pl.BlockSpec((block,), lambda i: (i,)),
        out_shape=jax.ShapeDtypeStruct((n,), jnp.float32),
    )(x, y)
    return out


# speedup vs baseline: 4.1916x; 1.0218x over previous
import jax
import jax.numpy as jnp
from jax.experimental import pallas as pl


def _stable_zero_div_body(x_ref, y_ref, o_ref):
    x = x_ref[...]
    y = y_ref[...]
    nz = y != 0.0
    inv = jnp.where(nz, 1.0 / jnp.where(nz, y, 1.0), 0.0)
    o_ref[...] = inv * x


def kernel(x, y):
    n = x.shape[0]
    block = 1048576
    out = pl.pallas_call(
        _stable_zero_div_body,
        grid=(n // block,),
        in_specs=[
            pl.BlockSpec((block,), lambda i: (i,)),
            pl.BlockSpec((block,), lambda i: (i,)),
        ],
        out_specs=pl.BlockSpec((block,), lambda i: (i,)),
        out_shape=jax.ShapeDtypeStruct((n,), jnp.float32),
    )(x, y)
    return out
